# Initial kernel scaffold; baseline (speedup 1.0000x reference)
import jax
import jax.numpy as jnp
from jax.experimental import pallas as pl


def kernel(x, edge_index, edge_attr, Wm1, bm1, Wa1, ba1, Wm2, bm2, Wa2, ba2):
    raise NotImplementedError("write your pallas kernel here")


# speedup vs baseline: 2.5137x; 2.5137x over previous
import functools

import jax
import jax.numpy as jnp
from jax import lax
from jax.experimental import pallas as pl
from jax.experimental.pallas import tpu as pltpu
from jax.experimental.pallas import tpu_sc as plsc

N = 10000
D = 256
H = 128
E = 160000
EPS = 0.1
NS = 16
EPW = E // NS
CHUNK = 80
NCH = EPW // CHUNK
NGRP = 5
GRP = NCH // NGRP
NP = 10240
ZR = 32
NZ = NP // NS // ZR
VPR = H // 16
RB = 1000
GRID = N // RB

_PREC = lax.Precision.HIGHEST


def _tc_pre_body(x_ref, w_ref, b_ref, u0_ref, u1_ref):
    u = jnp.dot(x_ref[...], w_ref[...], precision=_PREC,
                preferred_element_type=jnp.float32) + b_ref[...]
    u0_ref[...] = u[:, :H]
    u1_ref[...] = u[:, H:]


def _tc_pre(h, WmT, bm):
    return pl.pallas_call(
        _tc_pre_body,
        grid=(GRID,),
        in_specs=[
            pl.BlockSpec((RB, D), lambda i: (i, 0)),
            pl.BlockSpec((D, D), lambda i: (0, 0)),
            pl.BlockSpec((1, D), lambda i: (0, 0)),
        ],
        out_specs=[
            pl.BlockSpec((RB, H), lambda i: (i, 0)),
            pl.BlockSpec((RB, H), lambda i: (i, 0)),
        ],
        out_shape=[jax.ShapeDtypeStruct((N, H), jnp.float32)] * 2,
    )(h, WmT, bm)


def _tc_mid_body(a0_ref, a1_ref, x_ref, waT_ref, ba_ref, wmT_ref, bm_ref,
                 h1_ref, u0_ref, u1_ref):
    cat = jnp.concatenate([a0_ref[...], a1_ref[...], x_ref[...]], axis=1)
    o = jnp.dot(cat, waT_ref[...], precision=_PREC,
                preferred_element_type=jnp.float32) + ba_ref[...]
    h1 = x_ref[...] + EPS * jnp.maximum(o, 0.0)
    h1_ref[...] = h1
    u = jnp.dot(h1, wmT_ref[...], precision=_PREC,
                preferred_element_type=jnp.float32) + bm_ref[...]
    u0_ref[...] = u[:, :H]
    u1_ref[...] = u[:, H:]


def _tc_mid(a0, a1, x2, WaT, ba, WmT, bm):
    return pl.pallas_call(
        _tc_mid_body,
        grid=(GRID,),
        in_specs=[
            pl.BlockSpec((RB, H), lambda i: (i, 0)),
            pl.BlockSpec((RB, H), lambda i: (i, 0)),
            pl.BlockSpec((RB, D), lambda i: (i, 0)),
            pl.BlockSpec((2 * D, D), lambda i: (0, 0)),
            pl.BlockSpec((1, D), lambda i: (0, 0)),
            pl.BlockSpec((D, D), lambda i: (0, 0)),
            pl.BlockSpec((1, D), lambda i: (0, 0)),
        ],
        out_specs=[
            pl.BlockSpec((RB, D), lambda i: (i, 0)),
            pl.BlockSpec((RB, H), lambda i: (i, 0)),
            pl.BlockSpec((RB, H), lambda i: (i, 0)),
        ],
        out_shape=[
            jax.ShapeDtypeStruct((N, D), jnp.float32),
            jax.ShapeDtypeStruct((N, H), jnp.float32),
            jax.ShapeDtypeStruct((N, H), jnp.float32),
        ],
    )(a0, a1, x2, WaT, ba, WmT, bm)


def _tc_post_body(a0_ref, a1_ref, h_ref, waT_ref, ba_ref, out_ref):
    cat = jnp.concatenate([a0_ref[...], a1_ref[...], h_ref[...]], axis=1)
    o = jnp.dot(cat, waT_ref[...], precision=_PREC,
                preferred_element_type=jnp.float32) + ba_ref[...]
    out_ref[...] = jnp.maximum(o, 0.0)


def _tc_post(a0, a1, h1, WaT, ba):
    return pl.pallas_call(
        _tc_post_body,
        grid=(GRID,),
        in_specs=[
            pl.BlockSpec((RB, H), lambda i: (i, 0)),
            pl.BlockSpec((RB, H), lambda i: (i, 0)),
            pl.BlockSpec((RB, D), lambda i: (i, 0)),
            pl.BlockSpec((2 * D, D), lambda i: (0, 0)),
            pl.BlockSpec((1, D), lambda i: (0, 0)),
        ],
        out_specs=pl.BlockSpec((RB, D), lambda i: (i, 0)),
        out_shape=jax.ShapeDtypeStruct((N, D), jnp.float32),
    )(a0, a1, h1, WaT, ba)


@functools.partial(
    pl.kernel,
    out_type=[jax.ShapeDtypeStruct((NP, H), jnp.float32)] * 2,
    mesh=plsc.VectorSubcoreMesh(core_axis_name="c", subcore_axis_name="s"),
    scratch_types=[
        pltpu.VMEM((GRP, CHUNK), jnp.int32),
        pltpu.VMEM((GRP, CHUNK), jnp.int32),
        pltpu.VMEM((1, CHUNK * 16), jnp.float32),
        pltpu.VMEM((CHUNK, H), jnp.float32),
        pltpu.VMEM((1, H), jnp.float32),
        pltpu.VMEM((ZR, H), jnp.float32),
        pltpu.VMEM_SHARED((NP, H), jnp.float32),
        pltpu.SemaphoreType.DMA,
    ],
)
def _sc_edge(u0_hbm, u1_hbm, src_hbm, dst_hbm, eab_hbm, we_hbm,
             agg0_hbm, agg1_hbm,
             src_v, dst_v, eab_v, rows_v, we_v, zbuf, agg_sh, sem):
    c = lax.axis_index("c")
    s = lax.axis_index("s")

    def _zrow(i, carry):
        for j in range(VPR):
            zbuf[i, pl.ds(j * 16, 16)] = jnp.zeros((16,), jnp.float32)
        return carry

    lax.fori_loop(0, ZR, _zrow, 0)
    for t in range(NZ):
        pltpu.sync_copy(zbuf, agg_sh.at[pl.ds((s * NZ + t) * ZR, ZR)])

    pltpu.sync_copy(we_hbm.at[c], we_v)

    plsc.subcore_barrier()

    def _half(tbl_hbm, agg_hbm):
        wr = [we_v[0, pl.ds(j * 16, 16)] for j in range(VPR)]

        def _grp(g, carry):
            pltpu.sync_copy(src_hbm.at[s, g], src_v)
            pltpu.sync_copy(dst_hbm.at[s, g], dst_v)

            def _chunk(kk, c2):
                pltpu.sync_copy(eab_hbm.at[s, g * GRP + kk], eab_v)
                pltpu.async_copy(
                    tbl_hbm.at[src_v.at[kk]], rows_v, sem).wait()

                def _edge(i, cc):
                    eas = eab_v[0, pl.ds(i * 16, 16)]
                    for j in range(VPR):
                        v = rows_v[i, pl.ds(j * 16, 16)]
                        rows_v[i, pl.ds(j * 16, 16)] = jnp.maximum(
                            v + eas * wr[j], 0.0)
                    return cc

                lax.fori_loop(0, CHUNK, _edge, 0)
                pltpu.sync_copy(rows_v, agg_sh.at[dst_v.at[kk]], add=True)
                return c2

            lax.fori_loop(0, GRP, _chunk, 0)
            return carry

        lax.fori_loop(0, NGRP, _grp, 0)
        plsc.subcore_barrier()

        for t in range(NZ):
            sl = pl.ds((s * NZ + t) * ZR, ZR)
            pltpu.sync_copy(agg_sh.at[sl], agg_hbm.at[sl])

    @pl.when(c == 0)
    def _():
        _half(u0_hbm, agg0_hbm)

    @pl.when(c == 1)
    def _():
        _half(u1_hbm, agg1_hbm)


def kernel(x, edge_index, edge_attr, Wm1, bm1, Wa1, ba1, Wm2, bm2, Wa2, ba2):
    x2 = x[0]
    src = edge_index[0, 0].reshape(NS, NGRP, GRP, CHUNK)
    dst = edge_index[0, 1].reshape(NS, NGRP, GRP, CHUNK)
    eab = jnp.repeat(edge_attr[0][:, None], 16, axis=1).reshape(
        NS, NCH, 1, CHUNK * 16)

    Wm1T = Wm1[:, :D].T
    we1 = Wm1[:, D].reshape(2, 1, H)
    Wa1T = Wa1.T
    Wm2T = Wm2[:, :D].T
    we2 = Wm2[:, D].reshape(2, 1, H)
    Wa2T = Wa2.T
    bm1r = bm1.reshape(1, D)
    ba1r = ba1.reshape(1, D)
    bm2r = bm2.reshape(1, D)
    ba2r = ba2.reshape(1, D)

    u0, u1 = _tc_pre(x2, Wm1T, bm1r)
    a0, a1 = _sc_edge(u0, u1, src, dst, eab, we1)
    h1, v0, v1 = _tc_mid(a0, a1, x2, Wa1T, ba1r, Wm2T, bm2r)
    b0, b1 = _sc_edge(v0, v1, src, dst, eab, we2)
    out = _tc_post(b0, b1, h1, Wa2T, ba2r)
    return out[None]


# speedup vs baseline: 4.1658x; 1.6572x over previous
import functools

import jax
import jax.numpy as jnp
from jax import lax
from jax.experimental import pallas as pl
from jax.experimental.pallas import tpu as pltpu
from jax.experimental.pallas import tpu_sc as plsc

N = 10000
D = 256
H = 128
E = 160000
EPS = 0.1
NS = 16
EPW = E // NS
CHUNK = 100
NCH = EPW // CHUNK
NGRP = 5
GRP = NCH // NGRP
NP = 10240
ZR = 80
NZ = NP // NS // ZR
VPR = H // 16
RB = 1000
GRID = N // RB

_PREC = lax.Precision.HIGHEST


def _tc_pre_body(x_ref, w_ref, b_ref, u0_ref, u1_ref):
    u = jnp.dot(x_ref[...], w_ref[...], precision=_PREC,
                preferred_element_type=jnp.float32) + b_ref[...]
    u0_ref[...] = u[:, :H]
    u1_ref[...] = u[:, H:]


def _tc_pre(h, WmT, bm):
    return pl.pallas_call(
        _tc_pre_body,
        grid=(GRID,),
        in_specs=[
            pl.BlockSpec((RB, D), lambda i: (i, 0)),
            pl.BlockSpec((D, D), lambda i: (0, 0)),
            pl.BlockSpec((1, D), lambda i: (0, 0)),
        ],
        out_specs=[
            pl.BlockSpec((RB, H), lambda i: (i, 0)),
            pl.BlockSpec((RB, H), lambda i: (i, 0)),
        ],
        out_shape=[jax.ShapeDtypeStruct((N, H), jnp.float32)] * 2,
    )(h, WmT, bm)


def _tc_mid_body(a0_ref, a1_ref, x_ref, waT_ref, ba_ref, wmT_ref, bm_ref,
                 h1_ref, u0_ref, u1_ref):
    cat = jnp.concatenate([a0_ref[...], a1_ref[...], x_ref[...]], axis=1)
    o = jnp.dot(cat, waT_ref[...], precision=_PREC,
                preferred_element_type=jnp.float32) + ba_ref[...]
    h1 = x_ref[...] + EPS * jnp.maximum(o, 0.0)
    h1_ref[...] = h1
    u = jnp.dot(h1, wmT_ref[...], precision=_PREC,
                preferred_element_type=jnp.float32) + bm_ref[...]
    u0_ref[...] = u[:, :H]
    u1_ref[...] = u[:, H:]


def _tc_mid(a0, a1, x2, WaT, ba, WmT, bm):
    return pl.pallas_call(
        _tc_mid_body,
        grid=(GRID,),
        in_specs=[
            pl.BlockSpec((RB, H), lambda i: (i, 0)),
            pl.BlockSpec((RB, H), lambda i: (i, 0)),
            pl.BlockSpec((RB, D), lambda i: (i, 0)),
            pl.BlockSpec((2 * D, D), lambda i: (0, 0)),
            pl.BlockSpec((1, D), lambda i: (0, 0)),
            pl.BlockSpec((D, D), lambda i: (0, 0)),
            pl.BlockSpec((1, D), lambda i: (0, 0)),
        ],
        out_specs=[
            pl.BlockSpec((RB, D), lambda i: (i, 0)),
            pl.BlockSpec((RB, H), lambda i: (i, 0)),
            pl.BlockSpec((RB, H), lambda i: (i, 0)),
        ],
        out_shape=[
            jax.ShapeDtypeStruct((N, D), jnp.float32),
            jax.ShapeDtypeStruct((N, H), jnp.float32),
            jax.ShapeDtypeStruct((N, H), jnp.float32),
        ],
    )(a0, a1, x2, WaT, ba, WmT, bm)


def _tc_post_body(a0_ref, a1_ref, h_ref, waT_ref, ba_ref, out_ref):
    cat = jnp.concatenate([a0_ref[...], a1_ref[...], h_ref[...]], axis=1)
    o = jnp.dot(cat, waT_ref[...], precision=_PREC,
                preferred_element_type=jnp.float32) + ba_ref[...]
    out_ref[...] = jnp.maximum(o, 0.0)


def _tc_post(a0, a1, h1, WaT, ba):
    return pl.pallas_call(
        _tc_post_body,
        grid=(GRID,),
        in_specs=[
            pl.BlockSpec((RB, H), lambda i: (i, 0)),
            pl.BlockSpec((RB, H), lambda i: (i, 0)),
            pl.BlockSpec((RB, D), lambda i: (i, 0)),
            pl.BlockSpec((2 * D, D), lambda i: (0, 0)),
            pl.BlockSpec((1, D), lambda i: (0, 0)),
        ],
        out_specs=pl.BlockSpec((RB, D), lambda i: (i, 0)),
        out_shape=jax.ShapeDtypeStruct((N, D), jnp.float32),
    )(a0, a1, h1, WaT, ba)


@functools.partial(
    pl.kernel,
    out_type=[jax.ShapeDtypeStruct((NP, H), jnp.float32)] * 2,
    mesh=plsc.VectorSubcoreMesh(core_axis_name="c", subcore_axis_name="s"),
    scratch_types=[
        pltpu.VMEM((GRP, CHUNK), jnp.int32),
        pltpu.VMEM((GRP, CHUNK), jnp.int32),
        pltpu.VMEM((1, CHUNK * 16), jnp.float32),
        pltpu.VMEM((1, CHUNK * 16), jnp.float32),
        pltpu.VMEM((CHUNK, H), jnp.float32),
        pltpu.VMEM((CHUNK, H), jnp.float32),
        pltpu.VMEM((1, H), jnp.float32),
        pltpu.VMEM_SHARED((NP, H), jnp.float32),
        pltpu.SemaphoreType.DMA,
        pltpu.SemaphoreType.DMA,
        pltpu.SemaphoreType.DMA,
        pltpu.SemaphoreType.DMA,
    ],
)
def _sc_edge(u0_hbm, u1_hbm, src_hbm, dst_hbm, eab_hbm, we_hbm,
             agg0_hbm, agg1_hbm,
             src_v, dst_v, eab0, eab1, rows0, rows1, we_v, agg_sh,
             se0, se1, sg0, sg1):
    c = lax.axis_index("c")
    s = lax.axis_index("s")
    eabs, rows, ses, sgs = (eab0, eab1), (rows0, rows1), (se0, se1), (sg0, sg1)

    def _zrow(i, carry):
        for j in range(VPR):
            rows0[i, pl.ds(j * 16, 16)] = jnp.zeros((16,), jnp.float32)
        return carry

    lax.fori_loop(0, ZR, _zrow, 0)
    for t in range(NZ):
        pltpu.sync_copy(rows0.at[pl.ds(0, ZR)],
                        agg_sh.at[pl.ds((s * NZ + t) * ZR, ZR)])

    pltpu.sync_copy(we_hbm.at[c], we_v)

    plsc.subcore_barrier()

    def _half(tbl_hbm, agg_hbm):
        wr = [we_v[0, pl.ds(j * 16, 16)] for j in range(VPR)]

        def _issue(g, kk, b):
            pltpu.async_copy(eab_hbm.at[s, g * GRP + kk], eabs[b], ses[b])
            pltpu.async_copy(tbl_hbm.at[src_v.at[kk]], rows[b], sgs[b])

        def _wait(g, kk, b):
            pltpu.make_async_copy(
                eab_hbm.at[s, g * GRP + kk], eabs[b], ses[b]).wait()
            pltpu.make_async_copy(
                tbl_hbm.at[src_v.at[kk]], rows[b], sgs[b]).wait()

        def _grp(g, carry):
            pltpu.sync_copy(src_hbm.at[s, g], src_v)
            pltpu.sync_copy(dst_hbm.at[s, g], dst_v)
            _issue(g, 0, 0)

            def _pair(kp, c2):
                for half in range(2):
                    kk = 2 * kp + half
                    b = half
                    eab_b, rows_b = eabs[b], rows[b]

                    @pl.when(kk + 1 < GRP)
                    def _():
                        _issue(g, kk + 1, 1 - b)

                    _wait(g, kk, b)

                    def _edge(i, cc):
                        eas = eab_b[0, pl.ds(i * 16, 16)]
                        for j in range(VPR):
                            v = rows_b[i, pl.ds(j * 16, 16)]
                            rows_b[i, pl.ds(j * 16, 16)] = jnp.maximum(
                                v + eas * wr[j], 0.0)
                        return cc

                    lax.fori_loop(0, CHUNK, _edge, 0, unroll=2)
                    pltpu.sync_copy(rows_b, agg_sh.at[dst_v.at[kk]],
                                    add=True)
                return c2

            lax.fori_loop(0, GRP // 2, _pair, 0)
            return carry

        lax.fori_loop(0, NGRP, _grp, 0)
        plsc.subcore_barrier()

        for t in range(NZ):
            sl = pl.ds((s * NZ + t) * ZR, ZR)
            pltpu.sync_copy(agg_sh.at[sl], agg_hbm.at[sl])

    @pl.when(c == 0)
    def _():
        _half(u0_hbm, agg0_hbm)

    @pl.when(c == 1)
    def _():
        _half(u1_hbm, agg1_hbm)


def kernel(x, edge_index, edge_attr, Wm1, bm1, Wa1, ba1, Wm2, bm2, Wa2, ba2):
    x2 = x[0]
    src = edge_index[0, 0].reshape(NS, NGRP, GRP, CHUNK)
    dst = edge_index[0, 1].reshape(NS, NGRP, GRP, CHUNK)
    eab = jnp.repeat(edge_attr[0][:, None], 16, axis=1).reshape(
        NS, NCH, 1, CHUNK * 16)

    Wm1T = Wm1[:, :D].T
    we1 = Wm1[:, D].reshape(2, 1, H)
    Wa1T = Wa1.T
    Wm2T = Wm2[:, :D].T
    we2 = Wm2[:, D].reshape(2, 1, H)
    Wa2T = Wa2.T
    bm1r = bm1.reshape(1, D)
    ba1r = ba1.reshape(1, D)
    bm2r = bm2.reshape(1, D)
    ba2r = ba2.reshape(1, D)

    u0, u1 = _tc_pre(x2, Wm1T, bm1r)
    a0, a1 = _sc_edge(u0, u1, src, dst, eab, we1)
    h1, v0, v1 = _tc_mid(a0, a1, x2, Wa1T, ba1r, Wm2T, bm2r)
    b0, b1 = _sc_edge(v0, v1, src, dst, eab, we2)
    out = _tc_post(b0, b1, h1, Wa2T, ba2r)
    return out[None]


# speedup vs baseline: 4.1946x; 1.0069x over previous
import functools

import jax
import jax.numpy as jnp
from jax import lax
from jax.experimental import pallas as pl
from jax.experimental.pallas import tpu as pltpu
from jax.experimental.pallas import tpu_sc as plsc

N = 10000
D = 256
H = 128
E = 160000
EPS = 0.1
NS = 16
EPW = E // NS
CHUNK = 50
NCH = EPW // CHUNK
NGRP = 5
GRP = NCH // NGRP
TRI = (GRP - 1) // 3
NP = 10240
ZR = 40
NZ = NP // NS // ZR
VPR = H // 16
RB = 1000
GRID = N // RB

_PREC = lax.Precision.HIGHEST


def _tc_pre_body(x_ref, w_ref, b_ref, u0_ref, u1_ref):
    u = jnp.dot(x_ref[...], w_ref[...], precision=_PREC,
                preferred_element_type=jnp.float32) + b_ref[...]
    u0_ref[...] = u[:, :H]
    u1_ref[...] = u[:, H:]


def _tc_pre(h, WmT, bm):
    return pl.pallas_call(
        _tc_pre_body,
        grid=(GRID,),
        in_specs=[
            pl.BlockSpec((RB, D), lambda i: (i, 0)),
            pl.BlockSpec((D, D), lambda i: (0, 0)),
            pl.BlockSpec((1, D), lambda i: (0, 0)),
        ],
        out_specs=[
            pl.BlockSpec((RB, H), lambda i: (i, 0)),
            pl.BlockSpec((RB, H), lambda i: (i, 0)),
        ],
        out_shape=[jax.ShapeDtypeStruct((N, H), jnp.float32)] * 2,
    )(h, WmT, bm)


def _tc_mid_body(a0_ref, a1_ref, x_ref, waT_ref, ba_ref, wmT_ref, bm_ref,
                 h1_ref, u0_ref, u1_ref):
    cat = jnp.concatenate([a0_ref[...], a1_ref[...], x_ref[...]], axis=1)
    o = jnp.dot(cat, waT_ref[...], precision=_PREC,
                preferred_element_type=jnp.float32) + ba_ref[...]
    h1 = x_ref[...] + EPS * jnp.maximum(o, 0.0)
    h1_ref[...] = h1
    u = jnp.dot(h1, wmT_ref[...], precision=_PREC,
                preferred_element_type=jnp.float32) + bm_ref[...]
    u0_ref[...] = u[:, :H]
    u1_ref[...] = u[:, H:]


def _tc_mid(a0, a1, x2, WaT, ba, WmT, bm):
    return pl.pallas_call(
        _tc_mid_body,
        grid=(GRID,),
        in_specs=[
            pl.BlockSpec((RB, H), lambda i: (i, 0)),
            pl.BlockSpec((RB, H), lambda i: (i, 0)),
            pl.BlockSpec((RB, D), lambda i: (i, 0)),
            pl.BlockSpec((2 * D, D), lambda i: (0, 0)),
            pl.BlockSpec((1, D), lambda i: (0, 0)),
            pl.BlockSpec((D, D), lambda i: (0, 0)),
            pl.BlockSpec((1, D), lambda i: (0, 0)),
        ],
        out_specs=[
            pl.BlockSpec((RB, D), lambda i: (i, 0)),
            pl.BlockSpec((RB, H), lambda i: (i, 0)),
            pl.BlockSpec((RB, H), lambda i: (i, 0)),
        ],
        out_shape=[
            jax.ShapeDtypeStruct((N, D), jnp.float32),
            jax.ShapeDtypeStruct((N, H), jnp.float32),
            jax.ShapeDtypeStruct((N, H), jnp.float32),
        ],
    )(a0, a1, x2, WaT, ba, WmT, bm)


def _tc_post_body(a0_ref, a1_ref, h_ref, waT_ref, ba_ref, out_ref):
    cat = jnp.concatenate([a0_ref[...], a1_ref[...], h_ref[...]], axis=1)
    o = jnp.dot(cat, waT_ref[...], precision=_PREC,
                preferred_element_type=jnp.float32) + ba_ref[...]
    out_ref[...] = jnp.maximum(o, 0.0)


def _tc_post(a0, a1, h1, WaT, ba):
    return pl.pallas_call(
        _tc_post_body,
        grid=(GRID,),
        in_specs=[
            pl.BlockSpec((RB, H), lambda i: (i, 0)),
            pl.BlockSpec((RB, H), lambda i: (i, 0)),
            pl.BlockSpec((RB, D), lambda i: (i, 0)),
            pl.BlockSpec((2 * D, D), lambda i: (0, 0)),
            pl.BlockSpec((1, D), lambda i: (0, 0)),
        ],
        out_specs=pl.BlockSpec((RB, D), lambda i: (i, 0)),
        out_shape=jax.ShapeDtypeStruct((N, D), jnp.float32),
    )(a0, a1, h1, WaT, ba)


@functools.partial(
    pl.kernel,
    out_type=[jax.ShapeDtypeStruct((NP, H), jnp.float32)] * 2,
    mesh=plsc.VectorSubcoreMesh(core_axis_name="c", subcore_axis_name="s"),
    scratch_types=[
        pltpu.VMEM((GRP, CHUNK), jnp.int32),
        pltpu.VMEM((GRP, CHUNK), jnp.int32),
        pltpu.VMEM((1, CHUNK * 16), jnp.float32),
        pltpu.VMEM((1, CHUNK * 16), jnp.float32),
        pltpu.VMEM((1, CHUNK * 16), jnp.float32),
        pltpu.VMEM((CHUNK, H), jnp.float32),
        pltpu.VMEM((CHUNK, H), jnp.float32),
        pltpu.VMEM((CHUNK, H), jnp.float32),
        pltpu.VMEM((1, H), jnp.float32),
        pltpu.VMEM_SHARED((NP, H), jnp.float32),
        pltpu.SemaphoreType.DMA,
        pltpu.SemaphoreType.DMA,
        pltpu.SemaphoreType.DMA,
        pltpu.SemaphoreType.DMA,
        pltpu.SemaphoreType.DMA,
        pltpu.SemaphoreType.DMA,
        pltpu.SemaphoreType.DMA,
        pltpu.SemaphoreType.DMA,
        pltpu.SemaphoreType.DMA,
    ],
)
def _sc_edge(u0_hbm, u1_hbm, src_hbm, dst_hbm, eab_hbm, we_hbm,
             agg0_hbm, agg1_hbm,
             src_v, dst_v, eab0, eab1, eab2, rows0, rows1, rows2, we_v,
             agg_sh, se0, se1, se2, sg0, sg1, sg2, ss0, ss1, ss2):
    c = lax.axis_index("c")
    s = lax.axis_index("s")
    eabs, rows = (eab0, eab1, eab2), (rows0, rows1, rows2)
    ses, sgs, sss = (se0, se1, se2), (sg0, sg1, sg2), (ss0, ss1, ss2)

    def _zrow(i, carry):
        for j in range(VPR):
            rows0[i, pl.ds(j * 16, 16)] = jnp.zeros((16,), jnp.float32)
        return carry

    lax.fori_loop(0, ZR, _zrow, 0)
    for t in range(NZ):
        pltpu.sync_copy(rows0.at[pl.ds(0, ZR)],
                        agg_sh.at[pl.ds((s * NZ + t) * ZR, ZR)])

    pltpu.sync_copy(we_hbm.at[c], we_v)

    plsc.subcore_barrier()

    def _half(tbl_hbm, agg_hbm):
        wr = [we_v[0, pl.ds(j * 16, 16)] for j in range(VPR)]

        def _issue(g, kk, b):
            pltpu.async_copy(eab_hbm.at[s, g * GRP + kk], eabs[b], ses[b])
            pltpu.async_copy(tbl_hbm.at[src_v.at[kk]], rows[b], sgs[b])

        def _wait_in(g, kk, b):
            pltpu.make_async_copy(
                eab_hbm.at[s, g * GRP + kk], eabs[b], ses[b]).wait()
            pltpu.make_async_copy(
                tbl_hbm.at[src_v.at[kk]], rows[b], sgs[b]).wait()

        def _wait_sc(kk, b):
            pltpu.make_async_copy(
                rows[b], agg_sh.at[dst_v.at[kk]], sss[b]).wait()

        def _body(g, kk, b, prefetch):
            if prefetch:
                nb = (b + 1) % 3

                @pl.when(kk >= 2)
                def _():
                    _wait_sc(kk - 2, nb)

                _issue(g, kk + 1, nb)
            _wait_in(g, kk, b)
            eab_b, rows_b = eabs[b], rows[b]

            def _edge(i, cc):
                eas = eab_b[0, pl.ds(i * 16, 16)]
                for j in range(VPR):
                    v = rows_b[i, pl.ds(j * 16, 16)]
                    rows_b[i, pl.ds(j * 16, 16)] = jnp.maximum(
                        v + eas * wr[j], 0.0)
                return cc

            lax.fori_loop(0, CHUNK, _edge, 0, unroll=2)
            pltpu.async_copy(rows_b, agg_sh.at[dst_v.at[kk]], sss[b],
                             add=True)

        def _grp(g, carry):
            pltpu.sync_copy(src_hbm.at[s, g], src_v)
            pltpu.sync_copy(dst_hbm.at[s, g], dst_v)
            _issue(g, 0, 0)

            def _triple(kt, c2):
                for t in range(3):
                    _body(g, 3 * kt + t, t, True)
                return c2

            lax.fori_loop(0, TRI, _triple, 0)
            _body(g, GRP - 1, (GRP - 1) % 3, False)
            _wait_sc(GRP - 3, (GRP - 3) % 3)
            _wait_sc(GRP - 2, (GRP - 2) % 3)
            _wait_sc(GRP - 1, (GRP - 1) % 3)
            return carry

        lax.fori_loop(0, NGRP, _grp, 0)
        plsc.subcore_barrier()

        for t in range(NZ):
            sl = pl.ds((s * NZ + t) * ZR, ZR)
            pltpu.sync_copy(agg_sh.at[sl], agg_hbm.at[sl])

    @pl.when(c == 0)
    def _():
        _half(u0_hbm, agg0_hbm)

    @pl.when(c == 1)
    def _():
        _half(u1_hbm, agg1_hbm)


def kernel(x, edge_index, edge_attr, Wm1, bm1, Wa1, ba1, Wm2, bm2, Wa2, ba2):
    x2 = x[0]
    src = edge_index[0, 0].reshape(NS, NGRP, GRP, CHUNK)
    dst = edge_index[0, 1].reshape(NS, NGRP, GRP, CHUNK)
    eab = jnp.repeat(edge_attr[0][:, None], 16, axis=1).reshape(
        NS, NCH, 1, CHUNK * 16)

    Wm1T = Wm1[:, :D].T
    we1 = Wm1[:, D].reshape(2, 1, H)
    Wa1T = Wa1.T
    Wm2T = Wm2[:, :D].T
    we2 = Wm2[:, D].reshape(2, 1, H)
    Wa2T = Wa2.T
    bm1r = bm1.reshape(1, D)
    ba1r = ba1.reshape(1, D)
    bm2r = bm2.reshape(1, D)
    ba2r = ba2.reshape(1, D)

    u0, u1 = _tc_pre(x2, Wm1T, bm1r)
    a0, a1 = _sc_edge(u0, u1, src, dst, eab, we1)
    h1, v0, v1 = _tc_mid(a0, a1, x2, Wa1T, ba1r, Wm2T, bm2r)
    b0, b1 = _sc_edge(v0, v1, src, dst, eab, we2)
    out = _tc_post(b0, b1, h1, Wa2T, ba2r)
    return out[None]


# speedup vs baseline: 4.5188x; 1.0773x over previous
import functools

import jax
import jax.numpy as jnp
from jax import lax
from jax.experimental import pallas as pl
from jax.experimental.pallas import tpu as pltpu
from jax.experimental.pallas import tpu_sc as plsc

N = 10000
D = 256
H = 128
E = 160000
EPS = 0.1
NS = 16
EPW = E // NS
CHUNK = 50
NCH = EPW // CHUNK
NGRP = 5
GRP = NCH // NGRP
TRI = (GRP - 1) // 3
NP = 10240
ZR = 40
NZ = NP // NS // ZR
VPR = H // 16
RB = 1000
GRID = N // RB

_PREC = lax.Precision.HIGHEST


def _tc_pre_body(x_ref, w_ref, b_ref, u0_ref, u1_ref):
    u = jnp.dot(x_ref[...], w_ref[...], precision=_PREC,
                preferred_element_type=jnp.float32) + b_ref[...]
    u0_ref[...] = u[:, :H]
    u1_ref[...] = u[:, H:]


def _tc_pre(h, WmT, bm):
    return pl.pallas_call(
        _tc_pre_body,
        grid=(GRID,),
        in_specs=[
            pl.BlockSpec((RB, D), lambda i: (i, 0)),
            pl.BlockSpec((D, D), lambda i: (0, 0)),
            pl.BlockSpec((1, D), lambda i: (0, 0)),
        ],
        out_specs=[
            pl.BlockSpec((RB, H), lambda i: (i, 0)),
            pl.BlockSpec((RB, H), lambda i: (i, 0)),
        ],
        out_shape=[jax.ShapeDtypeStruct((N, H), jnp.float32)] * 2,
    )(h, WmT, bm)


def _tc_mid_body(a0_ref, a1_ref, x_ref, waT_ref, ba_ref, wmT_ref, bm_ref,
                 h1_ref, u0_ref, u1_ref):
    cat = jnp.concatenate([a0_ref[...], a1_ref[...], x_ref[...]], axis=1)
    o = jnp.dot(cat, waT_ref[...], precision=_PREC,
                preferred_element_type=jnp.float32) + ba_ref[...]
    h1 = x_ref[...] + EPS * jnp.maximum(o, 0.0)
    h1_ref[...] = h1
    u = jnp.dot(h1, wmT_ref[...], precision=_PREC,
                preferred_element_type=jnp.float32) + bm_ref[...]
    u0_ref[...] = u[:, :H]
    u1_ref[...] = u[:, H:]


def _tc_mid(a0, a1, x2, WaT, ba, WmT, bm):
    return pl.pallas_call(
        _tc_mid_body,
        grid=(GRID,),
        in_specs=[
            pl.BlockSpec((RB, H), lambda i: (i, 0)),
            pl.BlockSpec((RB, H), lambda i: (i, 0)),
            pl.BlockSpec((RB, D), lambda i: (i, 0)),
            pl.BlockSpec((2 * D, D), lambda i: (0, 0)),
            pl.BlockSpec((1, D), lambda i: (0, 0)),
            pl.BlockSpec((D, D), lambda i: (0, 0)),
            pl.BlockSpec((1, D), lambda i: (0, 0)),
        ],
        out_specs=[
            pl.BlockSpec((RB, D), lambda i: (i, 0)),
            pl.BlockSpec((RB, H), lambda i: (i, 0)),
            pl.BlockSpec((RB, H), lambda i: (i, 0)),
        ],
        out_shape=[
            jax.ShapeDtypeStruct((N, D), jnp.float32),
            jax.ShapeDtypeStruct((N, H), jnp.float32),
            jax.ShapeDtypeStruct((N, H), jnp.float32),
        ],
    )(a0, a1, x2, WaT, ba, WmT, bm)


def _tc_post_body(a0_ref, a1_ref, h_ref, waT_ref, ba_ref, out_ref):
    cat = jnp.concatenate([a0_ref[...], a1_ref[...], h_ref[...]], axis=1)
    o = jnp.dot(cat, waT_ref[...], precision=_PREC,
                preferred_element_type=jnp.float32) + ba_ref[...]
    out_ref[...] = jnp.maximum(o, 0.0)


def _tc_post(a0, a1, h1, WaT, ba):
    return pl.pallas_call(
        _tc_post_body,
        grid=(GRID,),
        in_specs=[
            pl.BlockSpec((RB, H), lambda i: (i, 0)),
            pl.BlockSpec((RB, H), lambda i: (i, 0)),
            pl.BlockSpec((RB, D), lambda i: (i, 0)),
            pl.BlockSpec((2 * D, D), lambda i: (0, 0)),
            pl.BlockSpec((1, D), lambda i: (0, 0)),
        ],
        out_specs=pl.BlockSpec((RB, D), lambda i: (i, 0)),
        out_shape=jax.ShapeDtypeStruct((N, D), jnp.float32),
    )(a0, a1, h1, WaT, ba)


@functools.partial(
    pl.kernel,
    out_type=[jax.ShapeDtypeStruct((NP, H), jnp.float32)] * 2,
    mesh=plsc.VectorSubcoreMesh(core_axis_name="c", subcore_axis_name="s"),
    scratch_types=[
        pltpu.VMEM((GRP, CHUNK), jnp.int32),
        pltpu.VMEM((GRP, CHUNK), jnp.int32),
        pltpu.VMEM((1, CHUNK * 16), jnp.float32),
        pltpu.VMEM((1, CHUNK * 16), jnp.float32),
        pltpu.VMEM((1, CHUNK * 16), jnp.float32),
        pltpu.VMEM((CHUNK, H), jnp.float32),
        pltpu.VMEM((CHUNK, H), jnp.float32),
        pltpu.VMEM((CHUNK, H), jnp.float32),
        pltpu.VMEM((1, H), jnp.float32),
        pltpu.VMEM_SHARED((NP, H), jnp.float32),
        pltpu.SemaphoreType.DMA,
        pltpu.SemaphoreType.DMA,
        pltpu.SemaphoreType.DMA,
        pltpu.SemaphoreType.DMA,
        pltpu.SemaphoreType.DMA,
        pltpu.SemaphoreType.DMA,
        pltpu.SemaphoreType.DMA,
        pltpu.SemaphoreType.DMA,
        pltpu.SemaphoreType.DMA,
    ],
)
def _sc_edge(u0_hbm, u1_hbm, src_hbm, dst_hbm, eab_hbm, we_hbm,
             agg0_hbm, agg1_hbm,
             src_v, dst_v, eab0, eab1, eab2, rows0, rows1, rows2, we_v,
             agg_sh, se0, se1, se2, sg0, sg1, sg2, ss0, ss1, ss2):
    c = lax.axis_index("c")
    s = lax.axis_index("s")
    eabs, rows = (eab0, eab1, eab2), (rows0, rows1, rows2)
    ses, sgs, sss = (se0, se1, se2), (sg0, sg1, sg2), (ss0, ss1, ss2)

    def _zrow(i, carry):
        for j in range(VPR):
            rows0[i, pl.ds(j * 16, 16)] = jnp.zeros((16,), jnp.float32)
        return carry

    lax.fori_loop(0, ZR, _zrow, 0)
    for t in range(NZ):
        pltpu.sync_copy(rows0.at[pl.ds(0, ZR)],
                        agg_sh.at[pl.ds((s * NZ + t) * ZR, ZR)])

    pltpu.sync_copy(we_hbm.at[c], we_v)

    plsc.subcore_barrier()

    def _half(tbl_hbm, agg_hbm):
        wr = [we_v[0, pl.ds(j * 16, 16)] for j in range(VPR)]

        def _issue(g, kk, b):
            pltpu.async_copy(eab_hbm.at[s, g * GRP + kk], eabs[b], ses[b])
            pltpu.async_copy(tbl_hbm.at[src_v.at[kk]], rows[b], sgs[b])

        def _wait_in(g, kk, b):
            pltpu.make_async_copy(
                eab_hbm.at[s, g * GRP + kk], eabs[b], ses[b]).wait()
            pltpu.make_async_copy(
                tbl_hbm.at[src_v.at[kk]], rows[b], sgs[b]).wait()

        def _wait_sc(kk, b):
            pltpu.make_async_copy(
                rows[b], agg_sh.at[dst_v.at[kk]], sss[b]).wait()

        def _body(g, kk, b, prefetch):
            if prefetch:
                nb = (b + 1) % 3

                @pl.when(kk >= 2)
                def _():
                    _wait_sc(kk - 2, nb)

                _issue(g, kk + 1, nb)
            _wait_in(g, kk, b)
            eab_b, rows_b = eabs[b], rows[b]

            @plsc.parallel_loop(0, CHUNK, unroll=5)
            def _edge(i):
                eas = eab_b[0, pl.ds(i * 16, 16)]
                for j in range(VPR):
                    v = rows_b[i, pl.ds(j * 16, 16)]
                    rows_b[i, pl.ds(j * 16, 16)] = jnp.maximum(
                        v + eas * wr[j], 0.0)
            pltpu.async_copy(rows_b, agg_sh.at[dst_v.at[kk]], sss[b],
                             add=True)

        def _grp(g, carry):
            pltpu.sync_copy(src_hbm.at[s, g], src_v)
            pltpu.sync_copy(dst_hbm.at[s, g], dst_v)
            _issue(g, 0, 0)

            def _triple(kt, c2):
                for t in range(3):
                    _body(g, 3 * kt + t, t, True)
                return c2

            lax.fori_loop(0, TRI, _triple, 0)
            _body(g, GRP - 1, (GRP - 1) % 3, False)
            _wait_sc(GRP - 3, (GRP - 3) % 3)
            _wait_sc(GRP - 2, (GRP - 2) % 3)
            _wait_sc(GRP - 1, (GRP - 1) % 3)
            return carry

        lax.fori_loop(0, NGRP, _grp, 0)
        plsc.subcore_barrier()

        for t in range(NZ):
            sl = pl.ds((s * NZ + t) * ZR, ZR)
            pltpu.sync_copy(agg_sh.at[sl], agg_hbm.at[sl])

    @pl.when(c == 0)
    def _():
        _half(u0_hbm, agg0_hbm)

    @pl.when(c == 1)
    def _():
        _half(u1_hbm, agg1_hbm)


def kernel(x, edge_index, edge_attr, Wm1, bm1, Wa1, ba1, Wm2, bm2, Wa2, ba2):
    x2 = x[0]
    src = edge_index[0, 0].reshape(NS, NGRP, GRP, CHUNK)
    dst = edge_index[0, 1].reshape(NS, NGRP, GRP, CHUNK)
    eab = jnp.repeat(edge_attr[0][:, None], 16, axis=1).reshape(
        NS, NCH, 1, CHUNK * 16)

    Wm1T = Wm1[:, :D].T
    we1 = Wm1[:, D].reshape(2, 1, H)
    Wa1T = Wa1.T
    Wm2T = Wm2[:, :D].T
    we2 = Wm2[:, D].reshape(2, 1, H)
    Wa2T = Wa2.T
    bm1r = bm1.reshape(1, D)
    ba1r = ba1.reshape(1, D)
    bm2r = bm2.reshape(1, D)
    ba2r = ba2.reshape(1, D)

    u0, u1 = _tc_pre(x2, Wm1T, bm1r)
    a0, a1 = _sc_edge(u0, u1, src, dst, eab, we1)
    h1, v0, v1 = _tc_mid(a0, a1, x2, Wa1T, ba1r, Wm2T, bm2r)
    b0, b1 = _sc_edge(v0, v1, src, dst, eab, we2)
    out = _tc_post(b0, b1, h1, Wa2T, ba2r)
    return out[None]


# speedup vs baseline: 4.8588x; 1.0752x over previous
import functools

import jax
import jax.numpy as jnp
from jax import lax
from jax.experimental import pallas as pl
from jax.experimental.pallas import tpu as pltpu
from jax.experimental.pallas import tpu_sc as plsc

N = 10000
D = 256
H = 128
E = 160000
EPS = 0.1
NS = 16
EPW = E // NS
CHUNK = 50
NCH = EPW // CHUNK
NGRP = 5
GRP = NCH // NGRP
TRI = (GRP - 1) // 3
NP = 10240
ZR = 40
NZ = NP // NS // ZR
VPR = H // 16
RB = 1000
GRID = N // RB

_PREC = lax.Precision.DEFAULT


def _tc_pre_body(x_ref, w_ref, b_ref, u0_ref, u1_ref):
    u = jnp.dot(x_ref[...], w_ref[...], precision=_PREC,
                preferred_element_type=jnp.float32) + b_ref[...]
    u0_ref[...] = u[:, :H]
    u1_ref[...] = u[:, H:]


def _tc_pre(h, WmT, bm):
    return pl.pallas_call(
        _tc_pre_body,
        grid=(GRID,),
        in_specs=[
            pl.BlockSpec((RB, D), lambda i: (i, 0)),
            pl.BlockSpec((D, D), lambda i: (0, 0)),
            pl.BlockSpec((1, D), lambda i: (0, 0)),
        ],
        out_specs=[
            pl.BlockSpec((RB, H), lambda i: (i, 0)),
            pl.BlockSpec((RB, H), lambda i: (i, 0)),
        ],
        out_shape=[jax.ShapeDtypeStruct((N, H), jnp.float32)] * 2,
    )(h, WmT, bm)


def _tc_mid_body(a0_ref, a1_ref, x_ref, waT_ref, ba_ref, wmT_ref, bm_ref,
                 h1_ref, u0_ref, u1_ref):
    cat = jnp.concatenate([a0_ref[...], a1_ref[...], x_ref[...]], axis=1)
    o = jnp.dot(cat, waT_ref[...], precision=_PREC,
                preferred_element_type=jnp.float32) + ba_ref[...]
    h1 = x_ref[...] + EPS * jnp.maximum(o, 0.0)
    h1_ref[...] = h1
    u = jnp.dot(h1, wmT_ref[...], precision=_PREC,
                preferred_element_type=jnp.float32) + bm_ref[...]
    u0_ref[...] = u[:, :H]
    u1_ref[...] = u[:, H:]


def _tc_mid(a0, a1, x2, WaT, ba, WmT, bm):
    return pl.pallas_call(
        _tc_mid_body,
        grid=(GRID,),
        in_specs=[
            pl.BlockSpec((RB, H), lambda i: (i, 0)),
            pl.BlockSpec((RB, H), lambda i: (i, 0)),
            pl.BlockSpec((RB, D), lambda i: (i, 0)),
            pl.BlockSpec((2 * D, D), lambda i: (0, 0)),
            pl.BlockSpec((1, D), lambda i: (0, 0)),
            pl.BlockSpec((D, D), lambda i: (0, 0)),
            pl.BlockSpec((1, D), lambda i: (0, 0)),
        ],
        out_specs=[
            pl.BlockSpec((RB, D), lambda i: (i, 0)),
            pl.BlockSpec((RB, H), lambda i: (i, 0)),
            pl.BlockSpec((RB, H), lambda i: (i, 0)),
        ],
        out_shape=[
            jax.ShapeDtypeStruct((N, D), jnp.float32),
            jax.ShapeDtypeStruct((N, H), jnp.float32),
            jax.ShapeDtypeStruct((N, H), jnp.float32),
        ],
    )(a0, a1, x2, WaT, ba, WmT, bm)


def _tc_post_body(a0_ref, a1_ref, h_ref, waT_ref, ba_ref, out_ref):
    cat = jnp.concatenate([a0_ref[...], a1_ref[...], h_ref[...]], axis=1)
    o = jnp.dot(cat, waT_ref[...], precision=_PREC,
                preferred_element_type=jnp.float32) + ba_ref[...]
    out_ref[...] = jnp.maximum(o, 0.0)


def _tc_post(a0, a1, h1, WaT, ba):
    return pl.pallas_call(
        _tc_post_body,
        grid=(GRID,),
        in_specs=[
            pl.BlockSpec((RB, H), lambda i: (i, 0)),
            pl.BlockSpec((RB, H), lambda i: (i, 0)),
            pl.BlockSpec((RB, D), lambda i: (i, 0)),
            pl.BlockSpec((2 * D, D), lambda i: (0, 0)),
            pl.BlockSpec((1, D), lambda i: (0, 0)),
        ],
        out_specs=pl.BlockSpec((RB, D), lambda i: (i, 0)),
        out_shape=jax.ShapeDtypeStruct((N, D), jnp.float32),
    )(a0, a1, h1, WaT, ba)


@functools.partial(
    pl.kernel,
    out_type=[jax.ShapeDtypeStruct((NP, H), jnp.float32)] * 2,
    mesh=plsc.VectorSubcoreMesh(core_axis_name="c", subcore_axis_name="s"),
    scratch_types=[
        pltpu.VMEM((GRP, CHUNK), jnp.int32),
        pltpu.VMEM((GRP, CHUNK), jnp.int32),
        pltpu.VMEM((1, CHUNK * 16), jnp.float32),
        pltpu.VMEM((1, CHUNK * 16), jnp.float32),
        pltpu.VMEM((1, CHUNK * 16), jnp.float32),
        pltpu.VMEM((CHUNK, H), jnp.float32),
        pltpu.VMEM((CHUNK, H), jnp.float32),
        pltpu.VMEM((CHUNK, H), jnp.float32),
        pltpu.VMEM((1, H), jnp.float32),
        pltpu.VMEM_SHARED((NP, H), jnp.float32),
        pltpu.SemaphoreType.DMA,
        pltpu.SemaphoreType.DMA,
        pltpu.SemaphoreType.DMA,
        pltpu.SemaphoreType.DMA,
        pltpu.SemaphoreType.DMA,
        pltpu.SemaphoreType.DMA,
        pltpu.SemaphoreType.DMA,
        pltpu.SemaphoreType.DMA,
        pltpu.SemaphoreType.DMA,
    ],
)
def _sc_edge(u0_hbm, u1_hbm, src_hbm, dst_hbm, eab_hbm, we_hbm,
             agg0_hbm, agg1_hbm,
             src_v, dst_v, eab0, eab1, eab2, rows0, rows1, rows2, we_v,
             agg_sh, se0, se1, se2, sg0, sg1, sg2, ss0, ss1, ss2):
    c = lax.axis_index("c")
    s = lax.axis_index("s")
    eabs, rows = (eab0, eab1, eab2), (rows0, rows1, rows2)
    ses, sgs, sss = (se0, se1, se2), (sg0, sg1, sg2), (ss0, ss1, ss2)

    def _zrow(i, carry):
        for j in range(VPR):
            rows0[i, pl.ds(j * 16, 16)] = jnp.zeros((16,), jnp.float32)
        return carry

    lax.fori_loop(0, ZR, _zrow, 0)
    for t in range(NZ):
        pltpu.sync_copy(rows0.at[pl.ds(0, ZR)],
                        agg_sh.at[pl.ds((s * NZ + t) * ZR, ZR)])

    pltpu.sync_copy(we_hbm.at[c], we_v)

    plsc.subcore_barrier()

    def _half(tbl_hbm, agg_hbm):
        wr = [we_v[0, pl.ds(j * 16, 16)] for j in range(VPR)]

        def _issue(g, kk, b):
            pltpu.async_copy(eab_hbm.at[s, g * GRP + kk], eabs[b], ses[b])
            pltpu.async_copy(tbl_hbm.at[src_v.at[kk]], rows[b], sgs[b])

        def _wait_in(g, kk, b):
            pltpu.make_async_copy(
                eab_hbm.at[s, g * GRP + kk], eabs[b], ses[b]).wait()
            pltpu.make_async_copy(
                tbl_hbm.at[src_v.at[kk]], rows[b], sgs[b]).wait()

        def _wait_sc(kk, b):
            pltpu.make_async_copy(
                rows[b], agg_sh.at[dst_v.at[kk]], sss[b]).wait()

        def _body(g, kk, b, prefetch):
            if prefetch:
                nb = (b + 1) % 3

                @pl.when(kk >= 2)
                def _():
                    _wait_sc(kk - 2, nb)

                _issue(g, kk + 1, nb)
            _wait_in(g, kk, b)
            eab_b, rows_b = eabs[b], rows[b]

            @plsc.parallel_loop(0, CHUNK, unroll=5)
            def _edge(i):
                eas = eab_b[0, pl.ds(i * 16, 16)]
                for j in range(VPR):
                    v = rows_b[i, pl.ds(j * 16, 16)]
                    rows_b[i, pl.ds(j * 16, 16)] = jnp.maximum(
                        v + eas * wr[j], 0.0)
            pltpu.async_copy(rows_b, agg_sh.at[dst_v.at[kk]], sss[b],
                             add=True)

        def _grp(g, carry):
            pltpu.sync_copy(src_hbm.at[s, g], src_v)
            pltpu.sync_copy(dst_hbm.at[s, g], dst_v)
            _issue(g, 0, 0)

            def _triple(kt, c2):
                for t in range(3):
                    _body(g, 3 * kt + t, t, True)
                return c2

            lax.fori_loop(0, TRI, _triple, 0)
            _body(g, GRP - 1, (GRP - 1) % 3, False)
            _wait_sc(GRP - 3, (GRP - 3) % 3)
            _wait_sc(GRP - 2, (GRP - 2) % 3)
            _wait_sc(GRP - 1, (GRP - 1) % 3)
            return carry

        lax.fori_loop(0, NGRP, _grp, 0)
        plsc.subcore_barrier()

        for t in range(NZ):
            sl = pl.ds((s * NZ + t) * ZR, ZR)
            pltpu.sync_copy(agg_sh.at[sl], agg_hbm.at[sl])

    @pl.when(c == 0)
    def _():
        _half(u0_hbm, agg0_hbm)

    @pl.when(c == 1)
    def _():
        _half(u1_hbm, agg1_hbm)


def kernel(x, edge_index, edge_attr, Wm1, bm1, Wa1, ba1, Wm2, bm2, Wa2, ba2):
    x2 = x[0]
    src = edge_index[0, 0].reshape(NS, NGRP, GRP, CHUNK)
    dst = edge_index[0, 1].reshape(NS, NGRP, GRP, CHUNK)
    eab = jnp.repeat(edge_attr[0][:, None], 16, axis=1).reshape(
        NS, NCH, 1, CHUNK * 16)

    Wm1T = Wm1[:, :D].T
    we1 = Wm1[:, D].reshape(2, 1, H)
    Wa1T = Wa1.T
    Wm2T = Wm2[:, :D].T
    we2 = Wm2[:, D].reshape(2, 1, H)
    Wa2T = Wa2.T
    bm1r = bm1.reshape(1, D)
    ba1r = ba1.reshape(1, D)
    bm2r = bm2.reshape(1, D)
    ba2r = ba2.reshape(1, D)

    u0, u1 = _tc_pre(x2, Wm1T, bm1r)
    a0, a1 = _sc_edge(u0, u1, src, dst, eab, we1)
    h1, v0, v1 = _tc_mid(a0, a1, x2, Wa1T, ba1r, Wm2T, bm2r)
    b0, b1 = _sc_edge(v0, v1, src, dst, eab, we2)
    out = _tc_post(b0, b1, h1, Wa2T, ba2r)
    return out[None]


# speedup vs baseline: 4.9582x; 1.0205x over previous
import functools

import jax
import jax.numpy as jnp
from jax import lax
from jax.experimental import pallas as pl
from jax.experimental.pallas import tpu as pltpu
from jax.experimental.pallas import tpu_sc as plsc

N = 10000
D = 256
H = 128
E = 160000
EPS = 0.1
NS = 16
EPW = E // NS
CHUNK = 50
NCH = EPW // CHUNK
NGRP = 5
GRP = NCH // NGRP
TRI = (GRP - 1) // 3
NP = 10240
ZR = 40
NZ = NP // NS // ZR
VPR = H // 16
RB = 1000
GRID = N // RB

_PREC = lax.Precision.DEFAULT


def _tc_pre_body(x_ref, w_ref, b_ref, u0_ref, u1_ref):
    u = jnp.dot(x_ref[...], w_ref[...], precision=_PREC,
                preferred_element_type=jnp.float32) + b_ref[...]
    u0_ref[...] = u[:, :H]
    u1_ref[...] = u[:, H:]


def _tc_pre(h, WmT, bm):
    return pl.pallas_call(
        _tc_pre_body,
        grid=(GRID,),
        in_specs=[
            pl.BlockSpec((RB, D), lambda i: (i, 0)),
            pl.BlockSpec((D, D), lambda i: (0, 0)),
            pl.BlockSpec((1, D), lambda i: (0, 0)),
        ],
        out_specs=[
            pl.BlockSpec((RB, H), lambda i: (i, 0)),
            pl.BlockSpec((RB, H), lambda i: (i, 0)),
        ],
        out_shape=[jax.ShapeDtypeStruct((N, H), jnp.float32)] * 2,
    )(h, WmT, bm)


def _tc_mid_body(a0_ref, a1_ref, x_ref, waT_ref, ba_ref, wmT_ref, bm_ref,
                 h1_ref, u0_ref, u1_ref):
    cat = jnp.concatenate([a0_ref[...], a1_ref[...], x_ref[...]], axis=1)
    o = jnp.dot(cat, waT_ref[...], precision=_PREC,
                preferred_element_type=jnp.float32) + ba_ref[...]
    h1 = x_ref[...] + EPS * jnp.maximum(o, 0.0)
    h1_ref[...] = h1
    u = jnp.dot(h1, wmT_ref[...], precision=_PREC,
                preferred_element_type=jnp.float32) + bm_ref[...]
    u0_ref[...] = u[:, :H]
    u1_ref[...] = u[:, H:]


def _tc_mid(a0, a1, x2, WaT, ba, WmT, bm):
    return pl.pallas_call(
        _tc_mid_body,
        grid=(GRID,),
        in_specs=[
            pl.BlockSpec((RB, H), lambda i: (i, 0)),
            pl.BlockSpec((RB, H), lambda i: (i, 0)),
            pl.BlockSpec((RB, D), lambda i: (i, 0)),
            pl.BlockSpec((2 * D, D), lambda i: (0, 0)),
            pl.BlockSpec((1, D), lambda i: (0, 0)),
            pl.BlockSpec((D, D), lambda i: (0, 0)),
            pl.BlockSpec((1, D), lambda i: (0, 0)),
        ],
        out_specs=[
            pl.BlockSpec((RB, D), lambda i: (i, 0)),
            pl.BlockSpec((RB, H), lambda i: (i, 0)),
            pl.BlockSpec((RB, H), lambda i: (i, 0)),
        ],
        out_shape=[
            jax.ShapeDtypeStruct((N, D), jnp.float32),
            jax.ShapeDtypeStruct((N, H), jnp.float32),
            jax.ShapeDtypeStruct((N, H), jnp.float32),
        ],
    )(a0, a1, x2, WaT, ba, WmT, bm)


def _tc_post_body(a0_ref, a1_ref, h_ref, waT_ref, ba_ref, out_ref):
    cat = jnp.concatenate([a0_ref[...], a1_ref[...], h_ref[...]], axis=1)
    o = jnp.dot(cat, waT_ref[...], precision=_PREC,
                preferred_element_type=jnp.float32) + ba_ref[...]
    out_ref[...] = jnp.maximum(o, 0.0)


def _tc_post(a0, a1, h1, WaT, ba):
    return pl.pallas_call(
        _tc_post_body,
        grid=(GRID,),
        in_specs=[
            pl.BlockSpec((RB, H), lambda i: (i, 0)),
            pl.BlockSpec((RB, H), lambda i: (i, 0)),
            pl.BlockSpec((RB, D), lambda i: (i, 0)),
            pl.BlockSpec((2 * D, D), lambda i: (0, 0)),
            pl.BlockSpec((1, D), lambda i: (0, 0)),
        ],
        out_specs=pl.BlockSpec((RB, D), lambda i: (i, 0)),
        out_shape=jax.ShapeDtypeStruct((N, D), jnp.float32),
    )(a0, a1, h1, WaT, ba)


@functools.partial(
    pl.kernel,
    out_type=[jax.ShapeDtypeStruct((NP, H), jnp.float32)] * 2,
    mesh=plsc.VectorSubcoreMesh(core_axis_name="c", subcore_axis_name="s"),
    scratch_types=[
        pltpu.VMEM((GRP, CHUNK), jnp.int32),
        pltpu.VMEM((GRP, CHUNK), jnp.int32),
        pltpu.VMEM((1, CHUNK * 16), jnp.float32),
        pltpu.VMEM((1, CHUNK * 16), jnp.float32),
        pltpu.VMEM((1, CHUNK * 16), jnp.float32),
        pltpu.VMEM((CHUNK, H), jnp.float32),
        pltpu.VMEM((CHUNK, H), jnp.float32),
        pltpu.VMEM((CHUNK, H), jnp.float32),
        pltpu.VMEM((1, H), jnp.float32),
        pltpu.VMEM_SHARED((NP, H), jnp.float32),
        pltpu.SemaphoreType.DMA,
        pltpu.SemaphoreType.DMA,
        pltpu.SemaphoreType.DMA,
        pltpu.SemaphoreType.DMA,
        pltpu.SemaphoreType.DMA,
        pltpu.SemaphoreType.DMA,
        pltpu.SemaphoreType.DMA,
        pltpu.SemaphoreType.DMA,
        pltpu.SemaphoreType.DMA,
    ],
)
def _sc_edge(u0_hbm, u1_hbm, src_hbm, dst_hbm, eab_hbm, we_hbm,
             agg0_hbm, agg1_hbm,
             src_v, dst_v, eab0, eab1, eab2, rows0, rows1, rows2, we_v,
             agg_sh, se0, se1, se2, sg0, sg1, sg2, ss0, ss1, ss2):
    c = lax.axis_index("c")
    s = lax.axis_index("s")
    eabs, rows = (eab0, eab1, eab2), (rows0, rows1, rows2)
    ses, sgs, sss = (se0, se1, se2), (sg0, sg1, sg2), (ss0, ss1, ss2)

    def _zrow(i, carry):
        for j in range(VPR):
            rows0[i, pl.ds(j * 16, 16)] = jnp.zeros((16,), jnp.float32)
        return carry

    lax.fori_loop(0, ZR, _zrow, 0)
    for t in range(NZ):
        pltpu.async_copy(rows0.at[pl.ds(0, ZR)],
                         agg_sh.at[pl.ds((s * NZ + t) * ZR, ZR)], se0)
    for t in range(NZ):
        pltpu.make_async_copy(
            rows0.at[pl.ds(0, ZR)],
            agg_sh.at[pl.ds((s * NZ + t) * ZR, ZR)], se0).wait()

    pltpu.sync_copy(we_hbm.at[c], we_v)

    plsc.subcore_barrier()

    def _half(tbl_hbm, agg_hbm):
        wr = [we_v[0, pl.ds(j * 16, 16)] for j in range(VPR)]

        def _issue(g, kk, b):
            pltpu.async_copy(eab_hbm.at[s, g * GRP + kk], eabs[b], ses[b])
            pltpu.async_copy(tbl_hbm.at[src_v.at[kk]], rows[b], sgs[b])

        def _wait_in(g, kk, b):
            pltpu.make_async_copy(
                eab_hbm.at[s, g * GRP + kk], eabs[b], ses[b]).wait()
            pltpu.make_async_copy(
                tbl_hbm.at[src_v.at[kk]], rows[b], sgs[b]).wait()

        def _wait_sc(kk, b):
            pltpu.make_async_copy(
                rows[b], agg_sh.at[dst_v.at[kk]], sss[b]).wait()

        def _body(g, kk, b, prefetch):
            if prefetch:
                nb = (b + 1) % 3

                @pl.when(kk >= 2)
                def _():
                    _wait_sc(kk - 2, nb)

                _issue(g, kk + 1, nb)
            _wait_in(g, kk, b)
            eab_b, rows_b = eabs[b], rows[b]

            @plsc.parallel_loop(0, CHUNK, unroll=5)
            def _edge(i):
                eas = eab_b[0, pl.ds(i * 16, 16)]
                for j in range(VPR):
                    v = rows_b[i, pl.ds(j * 16, 16)]
                    rows_b[i, pl.ds(j * 16, 16)] = jnp.maximum(
                        v + eas * wr[j], 0.0)
            pltpu.async_copy(rows_b, agg_sh.at[dst_v.at[kk]], sss[b],
                             add=True)

        def _grp(g, carry):
            pltpu.sync_copy(src_hbm.at[s, g], src_v)
            pltpu.sync_copy(dst_hbm.at[s, g], dst_v)
            _issue(g, 0, 0)

            def _triple(kt, c2):
                for t in range(3):
                    _body(g, 3 * kt + t, t, True)
                return c2

            lax.fori_loop(0, TRI, _triple, 0)
            _body(g, GRP - 1, (GRP - 1) % 3, False)
            _wait_sc(GRP - 3, (GRP - 3) % 3)
            _wait_sc(GRP - 2, (GRP - 2) % 3)
            _wait_sc(GRP - 1, (GRP - 1) % 3)
            return carry

        lax.fori_loop(0, NGRP, _grp, 0)
        plsc.subcore_barrier()

        sl = pl.ds(s * (NP // NS), NP // NS)
        pltpu.sync_copy(agg_sh.at[sl], agg_hbm.at[sl])

    @pl.when(c == 0)
    def _():
        _half(u0_hbm, agg0_hbm)

    @pl.when(c == 1)
    def _():
        _half(u1_hbm, agg1_hbm)


def kernel(x, edge_index, edge_attr, Wm1, bm1, Wa1, ba1, Wm2, bm2, Wa2, ba2):
    x2 = x[0]
    src = edge_index[0, 0].reshape(NS, NGRP, GRP, CHUNK)
    dst = edge_index[0, 1].reshape(NS, NGRP, GRP, CHUNK)
    eab = jnp.repeat(edge_attr[0][:, None], 16, axis=1).reshape(
        NS, NCH, 1, CHUNK * 16)

    Wm1T = Wm1[:, :D].T
    we1 = Wm1[:, D].reshape(2, 1, H)
    Wa1T = Wa1.T
    Wm2T = Wm2[:, :D].T
    we2 = Wm2[:, D].reshape(2, 1, H)
    Wa2T = Wa2.T
    bm1r = bm1.reshape(1, D)
    ba1r = ba1.reshape(1, D)
    bm2r = bm2.reshape(1, D)
    ba2r = ba2.reshape(1, D)

    u0, u1 = _tc_pre(x2, Wm1T, bm1r)
    a0, a1 = _sc_edge(u0, u1, src, dst, eab, we1)
    h1, v0, v1 = _tc_mid(a0, a1, x2, Wa1T, ba1r, Wm2T, bm2r)
    b0, b1 = _sc_edge(v0, v1, src, dst, eab, we2)
    out = _tc_post(b0, b1, h1, Wa2T, ba2r)
    return out[None]


# speedup vs baseline: 5.4003x; 1.0892x over previous
import functools

import jax
import jax.numpy as jnp
from jax import lax
from jax.experimental import pallas as pl
from jax.experimental.pallas import tpu as pltpu
from jax.experimental.pallas import tpu_sc as plsc

N = 10000
D = 256
H = 128
E = 160000
EPS = 0.1
NS = 16
EPW = E // NS
CHUNK = 80
NCH = EPW // CHUNK
NGRP = 5
GRP = NCH // NGRP
TRI = (GRP - 1) // 3
NP = 10240
ZR = 40
NZ = NP // NS // ZR
VPR = H // 16
RB = 1000
GRID = N // RB

_PREC = lax.Precision.DEFAULT


def _tc_pre_body(x_ref, w_ref, b_ref, u0_ref, u1_ref):
    u = jnp.dot(x_ref[...], w_ref[...], precision=_PREC,
                preferred_element_type=jnp.float32) + b_ref[...]
    u0_ref[...] = u[:, :H]
    u1_ref[...] = u[:, H:]


def _tc_pre(h, WmT, bm):
    return pl.pallas_call(
        _tc_pre_body,
        grid=(GRID,),
        in_specs=[
            pl.BlockSpec((RB, D), lambda i: (i, 0)),
            pl.BlockSpec((D, D), lambda i: (0, 0)),
            pl.BlockSpec((1, D), lambda i: (0, 0)),
        ],
        out_specs=[
            pl.BlockSpec((RB, H), lambda i: (i, 0)),
            pl.BlockSpec((RB, H), lambda i: (i, 0)),
        ],
        out_shape=[jax.ShapeDtypeStruct((N, H), jnp.float32)] * 2,
    )(h, WmT, bm)


def _tc_mid_body(a0_ref, a1_ref, x_ref, waT_ref, ba_ref, wmT_ref, bm_ref,
                 h1_ref, u0_ref, u1_ref):
    cat = jnp.concatenate([a0_ref[...], a1_ref[...], x_ref[...]], axis=1)
    o = jnp.dot(cat, waT_ref[...], precision=_PREC,
                preferred_element_type=jnp.float32) + ba_ref[...]
    h1 = x_ref[...] + EPS * jnp.maximum(o, 0.0)
    h1_ref[...] = h1
    u = jnp.dot(h1, wmT_ref[...], precision=_PREC,
                preferred_element_type=jnp.float32) + bm_ref[...]
    u0_ref[...] = u[:, :H]
    u1_ref[...] = u[:, H:]


def _tc_mid(a0, a1, x2, WaT, ba, WmT, bm):
    return pl.pallas_call(
        _tc_mid_body,
        grid=(GRID,),
        in_specs=[
            pl.BlockSpec((RB, H), lambda i: (i, 0)),
            pl.BlockSpec((RB, H), lambda i: (i, 0)),
            pl.BlockSpec((RB, D), lambda i: (i, 0)),
            pl.BlockSpec((2 * D, D), lambda i: (0, 0)),
            pl.BlockSpec((1, D), lambda i: (0, 0)),
            pl.BlockSpec((D, D), lambda i: (0, 0)),
            pl.BlockSpec((1, D), lambda i: (0, 0)),
        ],
        out_specs=[
            pl.BlockSpec((RB, D), lambda i: (i, 0)),
            pl.BlockSpec((RB, H), lambda i: (i, 0)),
            pl.BlockSpec((RB, H), lambda i: (i, 0)),
        ],
        out_shape=[
            jax.ShapeDtypeStruct((N, D), jnp.float32),
            jax.ShapeDtypeStruct((N, H), jnp.float32),
            jax.ShapeDtypeStruct((N, H), jnp.float32),
        ],
    )(a0, a1, x2, WaT, ba, WmT, bm)


def _tc_post_body(a0_ref, a1_ref, h_ref, waT_ref, ba_ref, out_ref):
    cat = jnp.concatenate([a0_ref[...], a1_ref[...], h_ref[...]], axis=1)
    o = jnp.dot(cat, waT_ref[...], precision=_PREC,
                preferred_element_type=jnp.float32) + ba_ref[...]
    out_ref[...] = jnp.maximum(o, 0.0)


def _tc_post(a0, a1, h1, WaT, ba):
    return pl.pallas_call(
        _tc_post_body,
        grid=(GRID,),
        in_specs=[
            pl.BlockSpec((RB, H), lambda i: (i, 0)),
            pl.BlockSpec((RB, H), lambda i: (i, 0)),
            pl.BlockSpec((RB, D), lambda i: (i, 0)),
            pl.BlockSpec((2 * D, D), lambda i: (0, 0)),
            pl.BlockSpec((1, D), lambda i: (0, 0)),
        ],
        out_specs=pl.BlockSpec((RB, D), lambda i: (i, 0)),
        out_shape=jax.ShapeDtypeStruct((N, D), jnp.float32),
    )(a0, a1, h1, WaT, ba)


@functools.partial(
    pl.kernel,
    out_type=[jax.ShapeDtypeStruct((NP, H), jnp.float32)] * 2,
    mesh=plsc.VectorSubcoreMesh(core_axis_name="c", subcore_axis_name="s"),
    scratch_types=[
        pltpu.VMEM((GRP, CHUNK), jnp.int32),
        pltpu.VMEM((GRP, CHUNK), jnp.int32),
        pltpu.VMEM((1, CHUNK * 16), jnp.float32),
        pltpu.VMEM((1, CHUNK * 16), jnp.float32),
        pltpu.VMEM((1, CHUNK * 16), jnp.float32),
        pltpu.VMEM((CHUNK, H), jnp.float32),
        pltpu.VMEM((CHUNK, H), jnp.float32),
        pltpu.VMEM((CHUNK, H), jnp.float32),
        pltpu.VMEM((1, H), jnp.float32),
        pltpu.VMEM_SHARED((NP, H), jnp.float32),
        pltpu.SemaphoreType.DMA,
        pltpu.SemaphoreType.DMA,
        pltpu.SemaphoreType.DMA,
        pltpu.SemaphoreType.DMA,
        pltpu.SemaphoreType.DMA,
        pltpu.SemaphoreType.DMA,
        pltpu.SemaphoreType.DMA,
        pltpu.SemaphoreType.DMA,
        pltpu.SemaphoreType.DMA,
    ],
)
def _sc_edge(u0_hbm, u1_hbm, src_hbm, dst_hbm, eab_hbm, we_hbm,
             agg0_hbm, agg1_hbm,
             src_v, dst_v, eab0, eab1, eab2, rows0, rows1, rows2, we_v,
             agg_sh, se0, se1, se2, sg0, sg1, sg2, ss0, ss1, ss2):
    c = lax.axis_index("c")
    s = lax.axis_index("s")
    eabs, rows = (eab0, eab1, eab2), (rows0, rows1, rows2)
    ses, sgs, sss = (se0, se1, se2), (sg0, sg1, sg2), (ss0, ss1, ss2)

    def _zrow(i, carry):
        for j in range(VPR):
            rows0[i, pl.ds(j * 16, 16)] = jnp.zeros((16,), jnp.float32)
        return carry

    lax.fori_loop(0, ZR, _zrow, 0)
    for t in range(NZ):
        pltpu.async_copy(rows0.at[pl.ds(0, ZR)],
                         agg_sh.at[pl.ds((s * NZ + t) * ZR, ZR)], se0)
    for t in range(NZ):
        pltpu.make_async_copy(
            rows0.at[pl.ds(0, ZR)],
            agg_sh.at[pl.ds((s * NZ + t) * ZR, ZR)], se0).wait()

    pltpu.sync_copy(we_hbm.at[c], we_v)

    plsc.subcore_barrier()

    def _half(tbl_hbm, agg_hbm):
        wr = [we_v[0, pl.ds(j * 16, 16)] for j in range(VPR)]

        def _issue(g, kk, b):
            pltpu.async_copy(eab_hbm.at[s, g * GRP + kk], eabs[b], ses[b])
            pltpu.async_copy(tbl_hbm.at[src_v.at[kk]], rows[b], sgs[b])

        def _wait_in(g, kk, b):
            pltpu.make_async_copy(
                eab_hbm.at[s, g * GRP + kk], eabs[b], ses[b]).wait()
            pltpu.make_async_copy(
                tbl_hbm.at[src_v.at[kk]], rows[b], sgs[b]).wait()

        def _wait_sc(kk, b):
            pltpu.make_async_copy(
                rows[b], agg_sh.at[dst_v.at[kk]], sss[b]).wait()

        def _body(g, kk, b, prefetch):
            if prefetch:
                nb = (b + 1) % 3

                @pl.when(kk >= 2)
                def _():
                    _wait_sc(kk - 2, nb)

                _issue(g, kk + 1, nb)
            _wait_in(g, kk, b)
            eab_b, rows_b = eabs[b], rows[b]

            @plsc.parallel_loop(0, CHUNK, unroll=5)
            def _edge(i):
                eas = eab_b[0, pl.ds(i * 16, 16)]
                for j in range(VPR):
                    v = rows_b[i, pl.ds(j * 16, 16)]
                    rows_b[i, pl.ds(j * 16, 16)] = jnp.maximum(
                        v + eas * wr[j], 0.0)
            pltpu.async_copy(rows_b, agg_sh.at[dst_v.at[kk]], sss[b],
                             add=True)

        def _grp(g, carry):
            pltpu.sync_copy(src_hbm.at[s, g], src_v)
            pltpu.sync_copy(dst_hbm.at[s, g], dst_v)
            _issue(g, 0, 0)

            def _triple(kt, c2):
                for t in range(3):
                    _body(g, 3 * kt + t, t, True)
                return c2

            lax.fori_loop(0, TRI, _triple, 0)
            _body(g, GRP - 1, (GRP - 1) % 3, False)
            _wait_sc(GRP - 3, (GRP - 3) % 3)
            _wait_sc(GRP - 2, (GRP - 2) % 3)
            _wait_sc(GRP - 1, (GRP - 1) % 3)
            return carry

        lax.fori_loop(0, NGRP, _grp, 0)
        plsc.subcore_barrier()

        sl = pl.ds(s * (NP // NS), NP // NS)
        pltpu.sync_copy(agg_sh.at[sl], agg_hbm.at[sl])

    @pl.when(c == 0)
    def _():
        _half(u0_hbm, agg0_hbm)

    @pl.when(c == 1)
    def _():
        _half(u1_hbm, agg1_hbm)


def kernel(x, edge_index, edge_attr, Wm1, bm1, Wa1, ba1, Wm2, bm2, Wa2, ba2):
    x2 = x[0]
    src = edge_index[0, 0].reshape(NS, NGRP, GRP, CHUNK)
    dst = edge_index[0, 1].reshape(NS, NGRP, GRP, CHUNK)
    eab = jnp.repeat(edge_attr[0][:, None], 16, axis=1).reshape(
        NS, NCH, 1, CHUNK * 16)

    Wm1T = Wm1[:, :D].T
    we1 = Wm1[:, D].reshape(2, 1, H)
    Wa1T = Wa1.T
    Wm2T = Wm2[:, :D].T
    we2 = Wm2[:, D].reshape(2, 1, H)
    Wa2T = Wa2.T
    bm1r = bm1.reshape(1, D)
    ba1r = ba1.reshape(1, D)
    bm2r = bm2.reshape(1, D)
    ba2r = ba2.reshape(1, D)

    u0, u1 = _tc_pre(x2, Wm1T, bm1r)
    a0, a1 = _sc_edge(u0, u1, src, dst, eab, we1)
    h1, v0, v1 = _tc_mid(a0, a1, x2, Wa1T, ba1r, Wm2T, bm2r)
    b0, b1 = _sc_edge(v0, v1, src, dst, eab, we2)
    out = _tc_post(b0, b1, h1, Wa2T, ba2r)
    return out[None]
